# Initial kernel scaffold; baseline (speedup 1.0000x reference)
import jax
import jax.numpy as jnp
from jax.experimental import pallas as pl


def kernel(x, params):
    raise NotImplementedError("write your pallas kernel here")


# speedup vs baseline: 15.1451x; 15.1451x over previous
import jax
import jax.numpy as jnp
from jax import lax
from jax.experimental import pallas as pl

NUM_EXPERTS = 16
CAPACITY = 64
LOAD_PENALTY = 2.0
DIVERSITY_TEMP = 2.0
ALPHA = 0.6
BN_EPS = 1e-5
B = 256
NCLS = 10


def _conv(x, w, b):
    y = lax.conv_general_dilated(x, w, (1, 1), 'SAME',
                                 dimension_numbers=('NCHW', 'OIHW', 'NCHW'))
    return y + b[None, :, None, None]


def _bn(x, g, b, m, v):
    return (x - m[None, :, None, None]) / jnp.sqrt(v[None, :, None, None] + BN_EPS) \
        * g[None, :, None, None] + b[None, :, None, None]


def _maxpool(x):
    return lax.reduce_window(x, -jnp.inf, lax.max, (1, 1, 2, 2), (1, 1, 2, 2), 'VALID')


def _trunk(x, p):
    h = jax.nn.relu(_bn(_conv(x, p['c1w'], p['c1b']), p['g1'], p['be1'], p['m1'], p['v1']))
    h = jax.nn.relu(_bn(_conv(h, p['c2w'], p['c2b']), p['g2'], p['be2'], p['m2'], p['v2']))
    h = _maxpool(h)
    h = jax.nn.relu(_bn(_conv(h, p['c3w'], p['c3b']), p['g3'], p['be3'], p['m3'], p['v3']))
    h = jax.nn.relu(_bn(_conv(h, p['c4w'], p['c4b']), p['g4'], p['be4'], p['m4'], p['v4']))
    h = _maxpool(h)
    return h.mean(axis=(2, 3))


def _heads(x, p):
    feats = _trunk(x, p)
    logits_e = jnp.einsum('bd,ecd->bec', feats, p['cls_w']) + p['cls_b'][None]
    probs = jax.nn.softmax(logits_e, axis=2)
    ent = -(probs * jnp.log(jnp.clip(probs, 1e-12))).sum(axis=2)
    confidence = -ent
    h = jax.nn.relu(jnp.einsum('bd,ehd->beh', feats, p['gW1']) + p['gb1'][None])
    es = (jnp.einsum('beh,eoh->beo', h, p['gW2']) + p['gb2'][None])[:, :, 0] / DIVERSITY_TEMP
    routing = ALPHA * es + (1.0 - ALPHA) * confidence - LOAD_PENALTY * p['ema'][None]
    return logits_e, routing


def _route_combine_body(rs_ref, rst_ref, lg_ref, final_ref, d_ref):
    rs = rs_ref[...]
    rst = rst_ref[...]
    lg = lg_ref[...]

    i_col = lax.broadcasted_iota(jnp.int32, (B, B), 0)
    k_row = lax.broadcasted_iota(jnp.int32, (B, B), 1)

    avail_col = jnp.ones((B, 1), jnp.float32)
    avail_row = jnp.ones((1, B), jnp.float32)
    sel_cols = []
    for j in range(4):
        s_col = rs[:, j:j + 1]
        s_row = rst[j:j + 1, :]
        beats = jnp.where((s_row > s_col) | ((s_row == s_col) & (k_row < i_col)),
                          1.0, 0.0)
        rank_col = jnp.sum(beats * avail_row, axis=1, keepdims=True)
        beats2 = jnp.where((s_col > s_row) | ((s_col == s_row) & (i_col < k_row)),
                           1.0, 0.0)
        rank_row = jnp.sum(beats2 * avail_col, axis=0, keepdims=True)
        sel_col = avail_col * jnp.where(rank_col < CAPACITY, 1.0, 0.0)
        sel_row = avail_row * jnp.where(rank_row < CAPACITY, 1.0, 0.0)
        avail_col = avail_col - sel_col
        avail_row = avail_row - sel_row
        sel_cols.append(sel_col)

    e_lane = lax.broadcasted_iota(jnp.int32, (B, NUM_EXPERTS), 1)
    Df = jnp.zeros((B, NUM_EXPERTS), jnp.float32)
    for j in range(4):
        Df = Df + sel_cols[j] * jnp.where(e_lane == j, 1.0, 0.0)
    d_ref[...] = Df

    active = rs * Df
    active = active - jnp.max(active, axis=1, keepdims=True)
    z = active + (Df - 1.0) * 1e9
    z = z - jnp.max(z, axis=1, keepdims=True)
    ez = jnp.exp(z)
    w = ez / jnp.sum(ez, axis=1, keepdims=True)

    rep_r = lax.broadcasted_iota(jnp.int32, (NUM_EXPERTS, NUM_EXPERTS * NCLS), 0)
    rep_c = lax.broadcasted_iota(jnp.int32, (NUM_EXPERTS, NUM_EXPERTS * NCLS), 1)
    R = jnp.where(rep_r == rep_c // NCLS, 1.0, 0.0)
    w_big = jnp.dot(w, R, preferred_element_type=jnp.float32)
    t = w_big * lg
    sum_r = lax.broadcasted_iota(jnp.int32, (NUM_EXPERTS * NCLS, NCLS), 0)
    sum_c = lax.broadcasted_iota(jnp.int32, (NUM_EXPERTS * NCLS, NCLS), 1)
    S = jnp.where(sum_r % NCLS == sum_c, 1.0, 0.0)
    final_ref[...] = jnp.dot(t, S, preferred_element_type=jnp.float32)


def _route_combine(rs, logits_e, interpret=False):
    rst = rs.T
    lg = logits_e.reshape(B, NUM_EXPERTS * NCLS)
    final, Df = pl.pallas_call(
        _route_combine_body,
        out_shape=(
            jax.ShapeDtypeStruct((B, NCLS), jnp.float32),
            jax.ShapeDtypeStruct((B, NUM_EXPERTS), jnp.float32),
        ),
        interpret=interpret,
    )(rs, rst, lg)
    return final, Df


def kernel(x, params):
    logits_e, rs = _heads(x, params)
    final, Df = _route_combine(rs, logits_e)
    return final, rs, Df.astype(bool)
